# baseline (speedup 1.0000x reference)
import functools

import jax
import jax.numpy as jnp
from jax import lax
from jax.experimental import pallas as pl
from jax.experimental.pallas import tpu as pltpu
from jax.experimental.pallas import tpu_sc as plsc


def _sc_gather(table4, q, n):
    nw = 32
    b_per_w = n // nw
    ch = 128
    nch = b_per_w // ch
    w = table4.shape[1]
    mesh = plsc.VectorSubcoreMesh(core_axis_name="c", subcore_axis_name="s")

    @functools.partial(
        pl.kernel,
        mesh=mesh,
        compiler_params=pltpu.CompilerParams(use_tc_tiling_on_sc=False),
        out_type=jax.ShapeDtypeStruct((n, w), jnp.float32),
        scratch_types=[
            pltpu.VMEM((b_per_w,), jnp.int32),
            pltpu.VMEM((b_per_w, w), jnp.float32),
            pltpu.SemaphoreType.DMA,
        ],
    )
    def k(table_hbm, q_hbm, out_hbm, idx_v, rows_v, sem):
        wid = lax.axis_index("s") * mesh.num_cores + lax.axis_index("c")
        base = wid * b_per_w
        pltpu.sync_copy(q_hbm.at[pl.ds(base, b_per_w)], idx_v)
        copies = [
            pltpu.async_copy(
                table_hbm.at[idx_v.at[pl.ds(j * ch, ch)]],
                rows_v.at[pl.ds(j * ch, ch)],
                sem,
            )
            for j in range(nch)
        ]
        for c in copies:
            c.wait()
        pltpu.sync_copy(rows_v, out_hbm.at[pl.ds(base, b_per_w), :])

    return k(table4, q)


def _tc_body(mus_ref, rf_ref, code_ref, conf_ref, out_ref):
    mu = mus_ref[...]
    rf = rf_ref[...]
    code = code_ref[...]
    conf = conf_ref[...]
    b = rf.shape[0]
    jj = lax.broadcasted_iota(jnp.int32, (128, 32), 0)
    kk = lax.broadcasted_iota(jnp.int32, (128, 32), 1)
    gmask = (jj // 32 == kk // 8).astype(jnp.float32)
    m4e = jnp.tile(jnp.exp(mu).T, (4, 4)) * gmask
    dn = (((1,), (1,)), ((), ()))
    p32 = lax.dot_general(jnp.exp(rf), m4e.T, dn,
                          preferred_element_type=jnp.float32)
    m = code >> 5
    ann = code & 31
    iota32 = lax.broadcasted_iota(jnp.int32, (b, 32), 1)
    oh32 = (iota32 == ann).astype(jnp.float32)
    mp = lax.dot_general(oh32, mu, dn,
                         preferred_element_type=jnp.float32)
    iota128 = lax.broadcasted_iota(jnp.int32, (b, 128), 1)
    rp = jnp.sum(jnp.where(iota128 == code, rf, 0.0),
                 axis=1, keepdims=True)
    p_pick = jnp.zeros((b, 8), jnp.float32)
    for g in range(4):
        p_pick = p_pick + jnp.where(m == g, p32[:, 8 * g:8 * (g + 1)], 0.0)
    ll = conf * (mp + rp - jnp.log(p_pick))
    out_ref[...] = ll.T


def _tc_compute(rf, mus, code_col, conf_col, b):
    n = rf.shape[0]
    c, d = mus.shape
    grid = n // b
    return pl.pallas_call(
        _tc_body,
        grid=(grid,),
        in_specs=[
            pl.BlockSpec((c, d), lambda i: (0, 0)),
            pl.BlockSpec((b, 4 * d), lambda i: (i, 0)),
            pl.BlockSpec((b, 1), lambda i: (i, 0)),
            pl.BlockSpec((b, 1), lambda i: (i, 0)),
        ],
        out_specs=pl.BlockSpec((c, b), lambda i: (0, i)),
        out_shape=jax.ShapeDtypeStruct((c, n), jnp.float32),
    )(mus, rf, code_col, conf_col)


def kernel(mus, random_effects, annotators, annotations, confidences):
    n = annotators.shape[0]
    v, d = random_effects.shape
    table4 = random_effects.reshape(v // 4, 4 * d)
    q = annotators >> 2
    rf = _sc_gather(table4, q, n)
    code = annotations + ((annotators & 3) << 5)
    code_col = code.reshape(n, 1)
    conf_col = confidences.reshape(n, 1)
    out_t = _tc_compute(rf, mus, code_col, conf_col, b=512)
    return out_t.T


# speedup vs baseline: 1.2567x; 1.2567x over previous
import functools

import jax
import jax.numpy as jnp
from jax import lax
from jax.experimental import pallas as pl
from jax.experimental.pallas import tpu as pltpu
from jax.experimental.pallas import tpu_sc as plsc


def _sc_gather(table, idx, n, d):
    nw = 32
    b_per_w = n // nw
    ch = 128
    nch = b_per_w // ch
    mesh = plsc.VectorSubcoreMesh(core_axis_name="c", subcore_axis_name="s")

    @functools.partial(
        pl.kernel,
        mesh=mesh,
        compiler_params=pltpu.CompilerParams(use_tc_tiling_on_sc=False),
        out_type=jax.ShapeDtypeStruct((n // 4, 4 * d), jnp.float32),
        scratch_types=[
            pltpu.VMEM((b_per_w,), jnp.int32),
            pltpu.VMEM((b_per_w, 128), jnp.float32),
            pltpu.SemaphoreType.DMA,
        ],
    )
    def k(table_hbm, idx_hbm, out_hbm, idx_v, rows_v, sem):
        wid = lax.axis_index("s") * mesh.num_cores + lax.axis_index("c")
        base = wid * b_per_w
        blk = wid // 4
        grp = wid % 4
        pltpu.sync_copy(idx_hbm.at[pl.ds(base, b_per_w)], idx_v)
        copies = [
            pltpu.async_copy(
                table_hbm.at[idx_v.at[pl.ds(j * ch, ch)]],
                rows_v.at[pl.ds(j * ch, ch)],
                sem,
            )
            for j in range(nch)
        ]
        for c in copies:
            c.wait()
        pltpu.sync_copy(
            rows_v.at[:, pl.ds(0, d)],
            out_hbm.at[pl.ds(blk * b_per_w, b_per_w), pl.ds(grp * d, d)],
        )

    return k(table, idx)


def _tc_body(mus_ref, rf_ref, ann_ref, conf_ref, out_ref):
    mu = mus_ref[...]
    rf = rf_ref[...]
    a_col = ann_ref[...]
    c_col = conf_ref[...]
    b = rf.shape[0]
    jj = lax.broadcasted_iota(jnp.int32, (128, 32), 0)
    kk = lax.broadcasted_iota(jnp.int32, (128, 32), 1)
    gmask = (jj // 32 == kk // 8).astype(jnp.float32)
    m4e = jnp.tile(jnp.exp(mu).T, (4, 4)) * gmask
    m4u = jnp.tile(mu.T, (4, 4)) * gmask
    dn = (((1,), (0,)), ((), ()))
    p4 = lax.dot_general(jnp.exp(rf), m4e, dn,
                         preferred_element_type=jnp.float32)
    iota32 = lax.broadcasted_iota(jnp.int32, (b, 32), 1)
    ohs, rps, cfs = [], [], []
    for g in range(4):
        ag = a_col[g * b:(g + 1) * b, :]
        oh = (iota32 == ag).astype(jnp.float32)
        ohs.append(oh)
        rg = rf[:, 32 * g:32 * (g + 1)]
        rps.append(jnp.sum(rg * oh, axis=1, keepdims=True))
        cfs.append(c_col[g * b:(g + 1) * b, :])
    oh128 = jnp.concatenate(ohs, axis=1)
    mp4 = lax.dot_general(oh128, m4u, dn,
                          preferred_element_type=jnp.float32)
    sum4_t = (mp4 - jnp.log(p4)).T
    rp_t = jnp.concatenate(rps, axis=1).T
    cf_t = jnp.concatenate(cfs, axis=1).T
    for g in range(4):
        out_ref[:, g * b:(g + 1) * b] = cf_t[g:g + 1, :] * (
            sum4_t[8 * g:8 * (g + 1), :] + rp_t[g:g + 1, :])


def _tc_compute(rf, mus, ann_col, conf_col, b):
    n4 = rf.shape[0]
    c, d = mus.shape
    grid = n4 // b
    return pl.pallas_call(
        _tc_body,
        grid=(grid,),
        in_specs=[
            pl.BlockSpec((c, d), lambda i: (0, 0)),
            pl.BlockSpec((b, 4 * d), lambda i: (i, 0)),
            pl.BlockSpec((4 * b, 1), lambda i: (i, 0)),
            pl.BlockSpec((4 * b, 1), lambda i: (i, 0)),
        ],
        out_specs=pl.BlockSpec((c, 4 * b), lambda i: (0, i)),
        out_shape=jax.ShapeDtypeStruct((c, 4 * n4), jnp.float32),
    )(mus, rf, ann_col, conf_col)


def kernel(mus, random_effects, annotators, annotations, confidences):
    n = annotators.shape[0]
    d = random_effects.shape[1]
    table128 = jnp.pad(random_effects, ((0, 0), (0, 128 - d)))
    rf = _sc_gather(table128, annotators, n, d)
    ann_col = annotations.reshape(n, 1)
    conf_col = confidences.reshape(n, 1)
    out_t = _tc_compute(rf, mus, ann_col, conf_col, b=512)
    return out_t.T


# speedup vs baseline: 1.6793x; 1.3363x over previous
import functools

import jax
import jax.numpy as jnp
from jax import lax
from jax.experimental import pallas as pl
from jax.experimental.pallas import tpu as pltpu
from jax.experimental.pallas import tpu_sc as plsc


def _sc_gather_t(table_t, idx, n, d):
    ch = 128
    nch = n // ch
    mesh = plsc.VectorSubcoreMesh(core_axis_name="c", subcore_axis_name="s")

    @functools.partial(
        pl.kernel,
        mesh=mesh,
        compiler_params=pltpu.CompilerParams(use_tc_tiling_on_sc=False),
        out_type=jax.ShapeDtypeStruct((d, n), jnp.float32),
        scratch_types=[
            pltpu.VMEM((n,), jnp.int32),
            pltpu.VMEM((n,), jnp.float32),
            pltpu.SemaphoreType.DMA,
        ],
    )
    def k(table_hbm, idx_hbm, out_hbm, idx_v, row_v, sem):
        wid = lax.axis_index("s") * mesh.num_cores + lax.axis_index("c")
        pltpu.sync_copy(idx_hbm, idx_v)
        src = table_hbm.at[wid]
        copies = [
            pltpu.async_copy(
                src.at[idx_v.at[pl.ds(j * ch, ch)]],
                row_v.at[pl.ds(j * ch, ch)],
                sem,
            )
            for j in range(nch)
        ]
        for c in copies:
            c.wait()
        pltpu.sync_copy(row_v, out_hbm.at[wid])

    return k(table_t, idx)


def _tc_body(mus_ref, rft_ref, ann_ref, conf_ref, out_ref):
    mu = mus_ref[...]
    rft = rft_ref[...]
    ann = ann_ref[...]
    conf = conf_ref[...]
    b = rft.shape[1]
    dn = (((1,), (0,)), ((), ()))
    p8 = lax.dot_general(jnp.exp(mu), jnp.exp(rft), dn,
                         preferred_element_type=jnp.float32)
    iota_d = lax.broadcasted_iota(jnp.int32, (32, b), 0)
    oht = (iota_d == ann).astype(jnp.float32)
    rpt = jnp.sum(rft * oht, axis=0, keepdims=True)
    mpt = lax.dot_general(mu, oht, dn,
                          preferred_element_type=jnp.float32)
    out_ref[...] = conf * (mpt + rpt - jnp.log(p8))


def _tc_compute(rft, mus, ann_row, conf_row, b):
    d, n = rft.shape
    c = mus.shape[0]
    grid = n // b
    return pl.pallas_call(
        _tc_body,
        grid=(grid,),
        in_specs=[
            pl.BlockSpec((c, d), lambda i: (0, 0)),
            pl.BlockSpec((d, b), lambda i: (0, i)),
            pl.BlockSpec((1, b), lambda i: (0, i)),
            pl.BlockSpec((1, b), lambda i: (0, i)),
        ],
        out_specs=pl.BlockSpec((c, b), lambda i: (0, i)),
        out_shape=jax.ShapeDtypeStruct((c, n), jnp.float32),
    )(mus, rft, ann_row, conf_row)


def kernel(mus, random_effects, annotators, annotations, confidences):
    n = annotators.shape[0]
    d = random_effects.shape[1]
    rft = _sc_gather_t(random_effects.T, annotators, n, d)
    ann_row = annotations.reshape(1, n)
    conf_row = confidences.reshape(1, n)
    out_t = _tc_compute(rft, mus, ann_row, conf_row, b=2048)
    return out_t.T


# speedup vs baseline: 1.7558x; 1.0455x over previous
import functools

import jax
import jax.numpy as jnp
from jax import lax
from jax.experimental import pallas as pl
from jax.experimental.pallas import tpu as pltpu
from jax.experimental.pallas import tpu_sc as plsc


def _sc_gather_t(table_t, idx, n, d):
    ch = 512
    nch = n // ch
    mesh = plsc.VectorSubcoreMesh(core_axis_name="c", subcore_axis_name="s")

    @functools.partial(
        pl.kernel,
        mesh=mesh,
        compiler_params=pltpu.CompilerParams(use_tc_tiling_on_sc=False),
        out_type=jax.ShapeDtypeStruct((d, n), jnp.float32),
        scratch_types=[
            pltpu.VMEM((n,), jnp.int32),
            pltpu.VMEM((n,), jnp.float32),
            pltpu.SemaphoreType.DMA,
        ],
    )
    def k(table_hbm, idx_hbm, out_hbm, idx_v, row_v, sem):
        wid = lax.axis_index("s") * mesh.num_cores + lax.axis_index("c")
        pltpu.sync_copy(idx_hbm, idx_v)
        src = table_hbm.at[wid]
        copies = [
            pltpu.async_copy(
                src.at[idx_v.at[pl.ds(j * ch, ch)]],
                row_v.at[pl.ds(j * ch, ch)],
                sem,
            )
            for j in range(nch)
        ]
        for c in copies:
            c.wait()
        pltpu.sync_copy(row_v, out_hbm.at[wid])

    return k(table_t, idx)


def _tc_body(mus_ref, rft_ref, ann_ref, conf_ref, out_ref):
    mu = mus_ref[...]
    rft = rft_ref[...]
    ann = ann_ref[...]
    conf = conf_ref[...]
    b = rft.shape[1]
    dn = (((1,), (0,)), ((), ()))
    p8 = lax.dot_general(jnp.exp(mu), jnp.exp(rft), dn,
                         preferred_element_type=jnp.float32)
    iota_d = lax.broadcasted_iota(jnp.int32, (32, b), 0)
    oht = (iota_d == ann).astype(jnp.float32)
    rpt = jnp.sum(rft * oht, axis=0, keepdims=True)
    mpt = lax.dot_general(mu, oht, dn,
                          preferred_element_type=jnp.float32)
    out_ref[...] = conf * (mpt + rpt - jnp.log(p8))


def _tc_compute(rft, mus, ann_row, conf_row, b):
    d, n = rft.shape
    c = mus.shape[0]
    grid = n // b
    return pl.pallas_call(
        _tc_body,
        grid=(grid,),
        in_specs=[
            pl.BlockSpec((c, d), lambda i: (0, 0)),
            pl.BlockSpec((d, b), lambda i: (0, i)),
            pl.BlockSpec((1, b), lambda i: (0, i)),
            pl.BlockSpec((1, b), lambda i: (0, i)),
        ],
        out_specs=pl.BlockSpec((c, b), lambda i: (0, i)),
        out_shape=jax.ShapeDtypeStruct((c, n), jnp.float32),
    )(mus, rft, ann_row, conf_row)


def kernel(mus, random_effects, annotators, annotations, confidences):
    n = annotators.shape[0]
    d = random_effects.shape[1]
    rft = _sc_gather_t(random_effects.T, annotators, n, d)
    ann_row = annotations.reshape(1, n)
    conf_row = confidences.reshape(1, n)
    out_t = _tc_compute(rft, mus, ann_row, conf_row, b=4096)
    return out_t.T


# speedup vs baseline: 1.7673x; 1.0065x over previous
import functools

import jax
import jax.numpy as jnp
from jax import lax
from jax.experimental import pallas as pl
from jax.experimental.pallas import tpu as pltpu
from jax.experimental.pallas import tpu_sc as plsc


def _sc_gather_t(table_t, idx, n, d):
    ch = n
    nch = n // ch
    mesh = plsc.VectorSubcoreMesh(core_axis_name="c", subcore_axis_name="s")

    @functools.partial(
        pl.kernel,
        mesh=mesh,
        compiler_params=pltpu.CompilerParams(use_tc_tiling_on_sc=False),
        out_type=jax.ShapeDtypeStruct((d, n), jnp.float32),
        scratch_types=[
            pltpu.VMEM((n,), jnp.int32),
            pltpu.VMEM((n,), jnp.float32),
            pltpu.SemaphoreType.DMA,
        ],
    )
    def k(table_hbm, idx_hbm, out_hbm, idx_v, row_v, sem):
        wid = lax.axis_index("s") * mesh.num_cores + lax.axis_index("c")
        pltpu.sync_copy(idx_hbm, idx_v)
        src = table_hbm.at[wid]
        copies = [
            pltpu.async_copy(
                src.at[idx_v.at[pl.ds(j * ch, ch)]],
                row_v.at[pl.ds(j * ch, ch)],
                sem,
            )
            for j in range(nch)
        ]
        for c in copies:
            c.wait()
        pltpu.sync_copy(row_v, out_hbm.at[wid])

    return k(table_t, idx)


def _tc_body(mus_ref, rft_ref, ann_ref, conf_ref, out_ref):
    mu = mus_ref[...]
    rft = rft_ref[...]
    ann = ann_ref[...]
    conf = conf_ref[...]
    b = rft.shape[1]
    dn = (((1,), (0,)), ((), ()))
    p8 = lax.dot_general(jnp.exp(mu), jnp.exp(rft), dn,
                         preferred_element_type=jnp.float32)
    iota_d = lax.broadcasted_iota(jnp.int32, (32, b), 0)
    oht = (iota_d == ann).astype(jnp.float32)
    rpt = jnp.sum(rft * oht, axis=0, keepdims=True)
    mpt = lax.dot_general(mu, oht, dn,
                          preferred_element_type=jnp.float32)
    out_ref[...] = conf * (mpt + rpt - jnp.log(p8))


def _tc_compute(rft, mus, ann_row, conf_row, b):
    d, n = rft.shape
    c = mus.shape[0]
    grid = n // b
    return pl.pallas_call(
        _tc_body,
        grid=(grid,),
        in_specs=[
            pl.BlockSpec((c, d), lambda i: (0, 0)),
            pl.BlockSpec((d, b), lambda i: (0, i)),
            pl.BlockSpec((1, b), lambda i: (0, i)),
            pl.BlockSpec((1, b), lambda i: (0, i)),
        ],
        out_specs=pl.BlockSpec((c, b), lambda i: (0, i)),
        out_shape=jax.ShapeDtypeStruct((c, n), jnp.float32),
    )(mus, rft, ann_row, conf_row)


def kernel(mus, random_effects, annotators, annotations, confidences):
    n = annotators.shape[0]
    d = random_effects.shape[1]
    rft = _sc_gather_t(random_effects.T, annotators, n, d)
    ann_row = annotations.reshape(1, n)
    conf_row = confidences.reshape(1, n)
    out_t = _tc_compute(rft, mus, ann_row, conf_row, b=4096)
    return out_t.T


# speedup vs baseline: 1.7990x; 1.0180x over previous
import functools

import jax
import jax.numpy as jnp
from jax import lax
from jax.experimental import pallas as pl
from jax.experimental.pallas import tpu as pltpu
from jax.experimental.pallas import tpu_sc as plsc


def _sc_gather_t(table_t, idx, n, d):
    ch = n
    nch = n // ch
    mesh = plsc.VectorSubcoreMesh(core_axis_name="c", subcore_axis_name="s")

    @functools.partial(
        pl.kernel,
        mesh=mesh,
        compiler_params=pltpu.CompilerParams(use_tc_tiling_on_sc=False),
        out_type=jax.ShapeDtypeStruct((d, n), jnp.float32),
        scratch_types=[
            pltpu.VMEM((n,), jnp.int32),
            pltpu.VMEM((n,), jnp.float32),
            pltpu.SemaphoreType.DMA,
        ],
    )
    def k(table_hbm, idx_hbm, out_hbm, idx_v, row_v, sem):
        wid = lax.axis_index("s") * mesh.num_cores + lax.axis_index("c")
        pltpu.sync_copy(idx_hbm, idx_v)
        src = table_hbm.at[wid]
        copies = [
            pltpu.async_copy(
                src.at[idx_v.at[pl.ds(j * ch, ch)]],
                row_v.at[pl.ds(j * ch, ch)],
                sem,
            )
            for j in range(nch)
        ]
        for c in copies:
            c.wait()
        pltpu.sync_copy(row_v, out_hbm.at[wid])

    return k(table_t, idx)


def _tc_body(mus_ref, rft_ref, ann_ref, conf_ref, out_ref):
    mu = mus_ref[...]
    rft = rft_ref[...]
    ann = ann_ref[...]
    conf = conf_ref[...]
    b = rft.shape[1]
    dn = (((1,), (0,)), ((), ()))
    p8 = lax.dot_general(jnp.exp(mu), jnp.exp(rft), dn,
                         preferred_element_type=jnp.float32)
    iota_d = lax.broadcasted_iota(jnp.int32, (32, b), 0)
    oht = (iota_d == ann).astype(jnp.float32)
    rpt = jnp.sum(rft * oht, axis=0, keepdims=True)
    mpt = lax.dot_general(mu, oht, dn,
                          preferred_element_type=jnp.float32)
    out_ref[...] = conf * (mpt + rpt - jnp.log(p8))


def _tc_compute(rft, mus, ann_row, conf_row, b):
    d, n = rft.shape
    c = mus.shape[0]
    grid = n // b
    return pl.pallas_call(
        _tc_body,
        grid=(grid,),
        in_specs=[
            pl.BlockSpec((c, d), lambda i: (0, 0)),
            pl.BlockSpec((d, b), lambda i: (0, i)),
            pl.BlockSpec((1, b), lambda i: (0, i)),
            pl.BlockSpec((1, b), lambda i: (0, i)),
        ],
        out_specs=pl.BlockSpec((c, b), lambda i: (0, i)),
        out_shape=jax.ShapeDtypeStruct((c, n), jnp.float32),
    )(mus, rft, ann_row, conf_row)


def kernel(mus, random_effects, annotators, annotations, confidences):
    n = annotators.shape[0]
    d = random_effects.shape[1]
    rft = _sc_gather_t(random_effects.T, annotators, n, d)
    ann_row = annotations.reshape(1, n)
    conf_row = confidences.reshape(1, n)
    out_t = _tc_compute(rft, mus, ann_row, conf_row, b=8192)
    return out_t.T
